# baseline (speedup 1.0000x reference)
import functools

import jax
import jax.numpy as jnp
from jax import lax
from jax.experimental import pallas as pl
from jax.experimental.pallas import tpu as pltpu
from jax.experimental.pallas import tpu_sc as plsc

_NC = 2
_NS = 16
_NW = _NC * _NS
_L = 16
_HUGE = 2**31 - 1


def _make(m, d, b, wc, shift):
    assert (1 << shift) == wc
    nfull = m // wc
    rem = m - nfull * wc
    rem_main = rem & ~127
    rem_tail = rem - rem_main
    tail_owner = nfull % _NW
    kfull = nfull // _NW
    nleft = nfull - kfull * _NW
    bw = b // _NW
    assert b % (_NW * _L) == 0 and d == 2 * _L

    mesh = plsc.VectorSubcoreMesh(core_axis_name="c", subcore_axis_name="s")

    out_types = [jax.ShapeDtypeStruct((d, m), jnp.float32)]
    if rem_tail:
        out_types.append(jax.ShapeDtypeStruct((rem_tail, d), jnp.float32))

    @functools.partial(
        pl.kernel,
        out_type=tuple(out_types),
        mesh=mesh,
        compiler_params=pltpu.CompilerParams(needs_layout_passes=False),
        scratch_types=[
            pltpu.VMEM((b + _L,), jnp.int32),
            pltpu.VMEM((b + _L,), jnp.int32),
            pltpu.VMEM((d,), jnp.float32),
            pltpu.VMEM((2 * _L,), jnp.int32),
            pltpu.VMEM((2 * _L,), jnp.int32),
            pltpu.VMEM((max(rem_tail, 1), d), jnp.float32),
            pltpu.VMEM((2, d, wc), jnp.float32),
            pltpu.VMEM((_L * d,), jnp.float32),
            pltpu.VMEM_SHARED((b * d,), jnp.float32),
            pltpu.SemaphoreType.DMA((2,)),
            pltpu.SemaphoreType.DMA((2,)),
            pltpu.SemaphoreType.DMA,
        ],
    )
    def scatter_kernel(xt_hbm, xtail_hbm, idx_hbm, valt_hbm, out_hbm, tail_hbm,
                       idx_v, myj_v, vrow_v, tmpj_v, tmpl_v, btail, bufs,
                       vslots, vals_sh, in_sems, out_sems, gsem):
        cid = lax.axis_index("c")
        sid = lax.axis_index("s")
        wid = sid * _NC + cid

        lanes = lax.iota(jnp.int32, _L)
        full = lanes >= 0

        def in_copy(wg, sl):
            return pltpu.make_async_copy(
                xt_hbm.at[:, pl.ds(wg * wc, wc)], bufs.at[sl],
                in_sems.at[sl])

        def out_copy(wg, sl):
            return pltpu.make_async_copy(
                bufs.at[sl], out_hbm.at[:, pl.ds(wg * wc, wc)],
                out_sems.at[sl])

        if kfull > 0:
            in_copy(wid, 0).start()

        def stage_values(tbuf, tbuf2):
            for p in range(2 * _NC):
                cbase = (sid * 2 * _NC + p) * (bw // 2)
                pltpu.sync_copy(valt_hbm.at[:, pl.ds(cbase, bw // 2)], tbuf)

                def tr_body(cc, _):
                    cv = jnp.full((_L,), cc, jnp.int32)
                    g0 = plsc.load_gather(tbuf, [lanes, cv])
                    g1 = plsc.load_gather(tbuf, [lanes + _L, cv])
                    tbuf2[pl.ds(cc * d, _L)] = g0
                    tbuf2[pl.ds(cc * d + _L, _L)] = g1
                    return 0

                lax.fori_loop(0, bw // 2, tr_body, 0, unroll=4)
                pltpu.sync_copy(tbuf2,
                                vals_sh.at[pl.ds(cbase * d, bw // 2 * d)])

        pl.run_scoped(stage_values,
                      pltpu.VMEM((d, bw // 2), jnp.float32),
                      pltpu.VMEM((bw // 2 * d,), jnp.float32))

        pltpu.sync_copy(idx_hbm, idx_v.at[pl.ds(0, b)])
        plsc.subcore_barrier()

        def scan_body(g, n):
            iv = idx_v[pl.ds(g * _L, _L)]
            q = lax.shift_right_logical(iv, shift)
            msk = (q & (_NW - 1)) == wid
            cnt = plsc.all_reduce_population_count(msk)[0]

            @pl.when(cnt > 0)
            def _():
                plsc.store_compressed(myj_v.at[pl.ds(n, _L)],
                                      lanes + g * _L, mask=msk)
                plsc.store_compressed(idx_v.at[pl.ds(n, _L)], iv, mask=msk)

            return n + cnt

        n = lax.fori_loop(0, b // _L, scan_body, jnp.int32(0), unroll=2)
        plsc.store_compressed(idx_v.at[pl.ds(n, _L)],
                              jnp.full((_L,), _HUGE, dtype=jnp.int32),
                              mask=full)
        nq = lax.div(n + _L - 1, _L)

        def apply_updates(pos_of, buf_store):
            def q_body(qi, _):
                lv = idx_v[pl.ds(qi * _L, _L)]
                pos, wm = pos_of(lv)
                c = plsc.all_reduce_population_count(wm)[0]

                @pl.when(c > 0)
                def _():
                    jv = myj_v[pl.ds(qi * _L, _L)]
                    plsc.store_compressed(tmpl_v.at[pl.ds(0, _L)], pos,
                                          mask=wm)
                    plsc.store_compressed(tmpj_v.at[pl.ds(0, _L)], jv,
                                          mask=wm)

                    def fetch(t, carry):
                        jt = tmpj_v[pl.ds(t, _L)][0]
                        pltpu.make_async_copy(
                            vals_sh.at[pl.ds(jt * d, d)],
                            vslots.at[pl.ds(t * d, d)], gsem).start()
                        return carry

                    lax.fori_loop(0, c, fetch, jnp.int32(0))

                    def drain(t, carry):
                        pltpu.make_async_copy(
                            vals_sh.at[pl.ds(0, d)],
                            vslots.at[pl.ds(0, d)], gsem).wait()
                        return carry

                    lax.fori_loop(0, c, drain, jnp.int32(0))

                    def item(t, carry):
                        pt = tmpl_v[pl.ds(t, _L)][0]
                        buf_store(pt, t)
                        return carry

                    lax.fori_loop(0, c, item, jnp.int32(0))

                return 0

            lax.fori_loop(0, nq, q_body, 0)

        def window_pos_of(wg):
            def pos_of(lv):
                wm = lax.shift_right_logical(lv, shift) == wg
                return lv & (wc - 1), wm
            return pos_of

        def main_windows():
            def col_store_in(sl):
                def col_store(pt, t):
                    posv = jnp.full((_L,), pt, jnp.int32)
                    slv = jnp.full((_L,), sl, jnp.int32)
                    g0 = plsc.load_gather(bufs, [slv, lanes, posv])
                    g1 = plsc.load_gather(bufs, [slv, lanes + _L, posv])
                    plsc.store_scatter(bufs, [slv, lanes, posv],
                                       g0 + vslots[pl.ds(t * d, _L)])
                    plsc.store_scatter(bufs, [slv, lanes + _L, posv],
                                       g1 + vslots[pl.ds(t * d + _L, _L)])
                return col_store

            def pipe_body(k, _):
                sl = k & 1
                wg = wid + _NW * k
                in_copy(wg, sl).wait()

                @pl.when(k + 1 < kfull)
                def _():
                    @pl.when(k >= 1)
                    def _():
                        out_copy(wg - _NW, 1 - sl).wait()

                    in_copy(wg + _NW, 1 - sl).start()

                apply_updates(window_pos_of(wg), col_store_in(sl))
                out_copy(wg, sl).start()
                return 0

            lax.fori_loop(0, kfull, pipe_body, 0)
            if kfull >= 2:
                out_copy(wid + _NW * (kfull - 2), kfull & 1).wait()
            if kfull >= 1:
                out_copy(wid + _NW * (kfull - 1), (kfull - 1) & 1).wait()

            if nleft:
                @pl.when(wid < nleft)
                def _():
                    wg = kfull * _NW + wid
                    pltpu.sync_copy(xt_hbm.at[:, pl.ds(wg * wc, wc)],
                                    bufs.at[0])
                    apply_updates(window_pos_of(wg), col_store_in(0))
                    pltpu.sync_copy(bufs.at[0],
                                    out_hbm.at[:, pl.ds(wg * wc, wc)])

            if rem_main:
                @pl.when(wid == tail_owner)
                def _():
                    base = nfull * wc
                    pltpu.sync_copy(xt_hbm.at[:, pl.ds(base, rem_main)],
                                    bufs.at[0, :, pl.ds(0, rem_main)])

                    def pos_of(lv):
                        wm = lax.shift_right_logical(lv, shift) == nfull
                        pos = lv & (wc - 1)
                        return pos, wm & (pos < rem_main)

                    apply_updates(pos_of, col_store_in(0))
                    pltpu.sync_copy(bufs.at[0, :, pl.ds(0, rem_main)],
                                    out_hbm.at[:, pl.ds(base, rem_main)])

        main_windows()

        if rem_tail:
            @pl.when(wid == tail_owner)
            def _():
                def row_store(pt, t):
                    for h in range(d // _L):
                        cur = btail[pt, pl.ds(h * _L, _L)]
                        btail[pt, pl.ds(h * _L, _L)] = (
                            cur + vslots[pl.ds(t * d + h * _L, _L)])

                pltpu.sync_copy(xtail_hbm, btail)

                def pos_of(lv):
                    wm = lax.shift_right_logical(lv, shift) == nfull
                    pos = (lv & (wc - 1)) - rem_main
                    return pos, wm & (pos >= 0)

                apply_updates(pos_of, row_store)
                pltpu.sync_copy(btail, tail_hbm)

    def run(x, indices, values):
        xt = jnp.swapaxes(x, 0, 1)
        vt = jnp.swapaxes(values, 0, 1)
        if rem_tail:
            xtail = lax.slice(x, (nfull * wc + rem_main, 0), (m, d))
            out_t, out_tail = scatter_kernel(xt, xtail, indices, vt)
            out = jnp.swapaxes(out_t, 0, 1)
            return lax.dynamic_update_slice(out, out_tail,
                                            (nfull * wc + rem_main, 0))
        (out_t,) = scatter_kernel(xt, indices, vt)
        return jnp.swapaxes(out_t, 0, 1)

    return run


def kernel(x, indices, values):
    m, d = x.shape
    b = indices.shape[0]
    fn = _make(m, d, b, wc=512, shift=9)
    return fn(x, indices, values)


# speedup vs baseline: 1.1825x; 1.1825x over previous
import functools

import jax
import jax.numpy as jnp
from jax import lax
from jax.experimental import pallas as pl
from jax.experimental.pallas import tpu as pltpu
from jax.experimental.pallas import tpu_sc as plsc

_NC = 2
_NS = 16
_NW = _NC * _NS
_L = 16
_HUGE = 2**31 - 1
_CAP = 4096


def _make(m, d, b, wc, shift):
    assert (1 << shift) == wc
    nfull = m // wc
    rem = m - nfull * wc
    rem_main = rem & ~127
    rem_tail = rem - rem_main
    tail_owner = nfull % _NW
    kfull = nfull // _NW
    nleft = nfull - kfull * _NW
    assert b % (_NW * _L) == 0 and d == 2 * _L and (b * d) % _NS == 0

    mesh = plsc.VectorSubcoreMesh(core_axis_name="c", subcore_axis_name="s")

    out_types = [jax.ShapeDtypeStruct((d, m), jnp.float32)]
    if rem_tail:
        out_types.append(jax.ShapeDtypeStruct((rem_tail, d), jnp.float32))

    @functools.partial(
        pl.kernel,
        out_type=tuple(out_types),
        mesh=mesh,
        compiler_params=pltpu.CompilerParams(needs_layout_passes=False),
        scratch_types=[
            pltpu.VMEM((b + _L,), jnp.int32),
            pltpu.VMEM((_CAP + _L,), jnp.int32),
            pltpu.VMEM((2 * _L,), jnp.int32),
            pltpu.VMEM((2 * _L,), jnp.int32),
            pltpu.VMEM((max(rem_tail, 1), d), jnp.float32),
            pltpu.VMEM((2, d, wc), jnp.float32),
            pltpu.VMEM((_L * d,), jnp.float32),
            pltpu.VMEM_SHARED((b * d,), jnp.float32),
            pltpu.SemaphoreType.DMA((2,)),
            pltpu.SemaphoreType.DMA((2,)),
            pltpu.SemaphoreType.DMA,
        ],
    )
    def scatter_kernel(xt_hbm, xtail_hbm, idx_hbm, vflat_hbm, out_hbm,
                       tail_hbm, idx_v, myj_v, tmpj_v, tmpl_v, btail, bufs,
                       vslots, vals_sh, in_sems, out_sems, gsem):
        cid = lax.axis_index("c")
        sid = lax.axis_index("s")
        wid = sid * _NC + cid

        lanes = lax.iota(jnp.int32, _L)
        full = lanes >= 0

        def in_copy(wg, sl):
            return pltpu.make_async_copy(
                xt_hbm.at[:, pl.ds(wg * wc, wc)], bufs.at[sl],
                in_sems.at[sl])

        def out_copy(wg, sl):
            return pltpu.make_async_copy(
                bufs.at[sl], out_hbm.at[:, pl.ds(wg * wc, wc)],
                out_sems.at[sl])

        if kfull > 0:
            in_copy(wid, 0).start()

        vseg = b * d // _NS
        pltpu.sync_copy(vflat_hbm.at[pl.ds(sid * vseg, vseg)],
                        vals_sh.at[pl.ds(sid * vseg, vseg)])

        pltpu.sync_copy(idx_hbm, idx_v.at[pl.ds(0, b)])
        plsc.subcore_barrier()

        def scan_body(g, n):
            iv = idx_v[pl.ds(g * _L, _L)]
            q = lax.shift_right_logical(iv, shift)
            msk = (q & (_NW - 1)) == wid
            cnt = plsc.all_reduce_population_count(msk)[0]

            @pl.when((cnt > 0) & (n + cnt <= _CAP))
            def _():
                plsc.store_compressed(myj_v.at[pl.ds(n, _L)],
                                      lanes + g * _L, mask=msk)
                plsc.store_compressed(idx_v.at[pl.ds(n, _L)], iv, mask=msk)

            return n + cnt

        n = lax.fori_loop(0, b // _L, scan_body, jnp.int32(0), unroll=2)
        fb = n > _CAP

        @pl.when(~fb)
        def _():
            plsc.store_compressed(idx_v.at[pl.ds(n, _L)],
                                  jnp.full((_L,), _HUGE, dtype=jnp.int32),
                                  mask=full)

        @pl.when(fb)
        def _():
            pltpu.sync_copy(idx_hbm, idx_v.at[pl.ds(0, b)])

        nq = lax.div(n + _L - 1, _L)

        def apply_updates(pos_of, buf_store):
            def make_q_body(jv_of):
                def q_body(qi, _):
                    lv = idx_v[pl.ds(qi * _L, _L)]
                    pos, wm = pos_of(lv)
                    c = plsc.all_reduce_population_count(wm)[0]

                    @pl.when(c > 0)
                    def _():
                        plsc.store_compressed(tmpl_v.at[pl.ds(0, _L)], pos,
                                              mask=wm)
                        plsc.store_compressed(tmpj_v.at[pl.ds(0, _L)],
                                              jv_of(qi), mask=wm)

                        def fetch(t, carry):
                            jt = tmpj_v[pl.ds(t, _L)][0]
                            pltpu.make_async_copy(
                                vals_sh.at[pl.ds(jt * d, d)],
                                vslots.at[pl.ds(t * d, d)], gsem).start()
                            return carry

                        lax.fori_loop(0, c, fetch, jnp.int32(0))

                        def drain(t, carry):
                            pltpu.make_async_copy(
                                vals_sh.at[pl.ds(0, d)],
                                vslots.at[pl.ds(0, d)], gsem).wait()
                            return carry

                        lax.fori_loop(0, c, drain, jnp.int32(0))

                        def item(t, carry):
                            pt = tmpl_v[pl.ds(t, _L)][0]
                            buf_store(pt, t)
                            return carry

                        lax.fori_loop(0, c, item, jnp.int32(0))

                    return 0

                return q_body

            @pl.when(~fb)
            def _():
                lax.fori_loop(0, nq,
                              make_q_body(
                                  lambda qi: myj_v[pl.ds(qi * _L, _L)]),
                              0)

            @pl.when(fb)
            def _():
                lax.fori_loop(0, b // _L,
                              make_q_body(lambda qi: lanes + qi * _L),
                              0)

        def window_pos_of(wg):
            def pos_of(lv):
                wm = lax.shift_right_logical(lv, shift) == wg
                return lv & (wc - 1), wm
            return pos_of

        def col_store_in(sl):
            def col_store(pt, t):
                posv = jnp.full((_L,), pt, jnp.int32)
                slv = jnp.full((_L,), sl, jnp.int32)
                g0 = plsc.load_gather(bufs, [slv, lanes, posv])
                g1 = plsc.load_gather(bufs, [slv, lanes + _L, posv])
                plsc.store_scatter(bufs, [slv, lanes, posv],
                                   g0 + vslots[pl.ds(t * d, _L)])
                plsc.store_scatter(bufs, [slv, lanes + _L, posv],
                                   g1 + vslots[pl.ds(t * d + _L, _L)])
            return col_store

        def pipe_body(k, _):
            sl = k & 1
            wg = wid + _NW * k
            in_copy(wg, sl).wait()

            @pl.when(k + 1 < kfull)
            def _():
                @pl.when(k >= 1)
                def _():
                    out_copy(wg - _NW, 1 - sl).wait()

                in_copy(wg + _NW, 1 - sl).start()

            apply_updates(window_pos_of(wg), col_store_in(sl))
            out_copy(wg, sl).start()
            return 0

        lax.fori_loop(0, kfull, pipe_body, 0)
        if kfull >= 2:
            out_copy(wid + _NW * (kfull - 2), kfull & 1).wait()
        if kfull >= 1:
            out_copy(wid + _NW * (kfull - 1), (kfull - 1) & 1).wait()

        if nleft:
            @pl.when(wid < nleft)
            def _():
                wg = kfull * _NW + wid
                pltpu.sync_copy(xt_hbm.at[:, pl.ds(wg * wc, wc)], bufs.at[0])
                apply_updates(window_pos_of(wg), col_store_in(0))
                pltpu.sync_copy(bufs.at[0],
                                out_hbm.at[:, pl.ds(wg * wc, wc)])

        if rem_main:
            @pl.when(wid == tail_owner)
            def _():
                base = nfull * wc
                pltpu.sync_copy(xt_hbm.at[:, pl.ds(base, rem_main)],
                                bufs.at[0, :, pl.ds(0, rem_main)])

                def pos_of(lv):
                    wm = lax.shift_right_logical(lv, shift) == nfull
                    pos = lv & (wc - 1)
                    return pos, wm & (pos < rem_main)

                apply_updates(pos_of, col_store_in(0))
                pltpu.sync_copy(bufs.at[0, :, pl.ds(0, rem_main)],
                                out_hbm.at[:, pl.ds(base, rem_main)])

        if rem_tail:
            @pl.when(wid == tail_owner)
            def _():
                def row_store(pt, t):
                    for h in range(d // _L):
                        cur = btail[pt, pl.ds(h * _L, _L)]
                        btail[pt, pl.ds(h * _L, _L)] = (
                            cur + vslots[pl.ds(t * d + h * _L, _L)])

                pltpu.sync_copy(xtail_hbm, btail)

                def pos_of(lv):
                    wm = lax.shift_right_logical(lv, shift) == nfull
                    pos = (lv & (wc - 1)) - rem_main
                    return pos, wm & (pos >= 0)

                apply_updates(pos_of, row_store)
                pltpu.sync_copy(btail, tail_hbm)

    def run(x, indices, values):
        xt = jnp.swapaxes(x, 0, 1)
        vflat = values.reshape(-1)
        if rem_tail:
            xtail = lax.slice(x, (nfull * wc + rem_main, 0), (m, d))
            out_t, out_tail = scatter_kernel(xt, xtail, indices, vflat)
            out = jnp.swapaxes(out_t, 0, 1)
            return lax.dynamic_update_slice(out, out_tail,
                                            (nfull * wc + rem_main, 0))
        (out_t,) = scatter_kernel(xt, indices, vflat)
        return jnp.swapaxes(out_t, 0, 1)

    return run


def kernel(x, indices, values):
    m, d = x.shape
    b = indices.shape[0]
    fn = _make(m, d, b, wc=1024, shift=10)
    return fn(x, indices, values)


# speedup vs baseline: 1.2203x; 1.0320x over previous
import functools

import jax
import jax.numpy as jnp
from jax import lax
from jax.experimental import pallas as pl
from jax.experimental.pallas import tpu as pltpu
from jax.experimental.pallas import tpu_sc as plsc

_NC = 2
_NS = 16
_NW = _NC * _NS
_L = 16
_HUGE = 2**31 - 1
_CAP = 4096


def _make(m, d, b, wc, shift):
    assert (1 << shift) == wc
    nfull = m // wc
    rem = m - nfull * wc
    rem_main = rem & ~127
    rem_tail = rem - rem_main
    tail_owner = nfull % _NW
    kfull = nfull // _NW
    nleft = nfull - kfull * _NW
    assert b % (_NW * _L) == 0 and d == 2 * _L and (b * d) % _NS == 0

    mesh = plsc.VectorSubcoreMesh(core_axis_name="c", subcore_axis_name="s")

    out_types = [jax.ShapeDtypeStruct((d, m), jnp.float32)]
    if rem_tail:
        out_types.append(jax.ShapeDtypeStruct((rem_tail, d), jnp.float32))

    @functools.partial(
        pl.kernel,
        out_type=tuple(out_types),
        mesh=mesh,
        compiler_params=pltpu.CompilerParams(needs_layout_passes=False),
        scratch_types=[
            pltpu.VMEM((b + _L,), jnp.int32),
            pltpu.VMEM((_CAP + _L,), jnp.int32),
            pltpu.VMEM((2 * _L,), jnp.int32),
            pltpu.VMEM((2 * _L,), jnp.int32),
            pltpu.VMEM((max(rem_tail, 1), d), jnp.float32),
            pltpu.VMEM((4, d, wc), jnp.float32),
            pltpu.VMEM((_L * d,), jnp.float32),
            pltpu.VMEM_SHARED((b * d,), jnp.float32),
            pltpu.SemaphoreType.DMA((4,)),
            pltpu.SemaphoreType.DMA((4,)),
            pltpu.SemaphoreType.DMA,
        ],
    )
    def scatter_kernel(xt_hbm, xtail_hbm, idx_hbm, vflat_hbm, out_hbm,
                       tail_hbm, idx_v, myj_v, tmpj_v, tmpl_v, btail, bufs,
                       vslots, vals_sh, in_sems, out_sems, gsem):
        cid = lax.axis_index("c")
        sid = lax.axis_index("s")
        wid = sid * _NC + cid

        lanes = lax.iota(jnp.int32, _L)
        full = lanes >= 0

        def in_copy(wg, sl):
            return pltpu.make_async_copy(
                xt_hbm.at[:, pl.ds(wg * wc, wc)], bufs.at[sl],
                in_sems.at[sl])

        def out_copy(wg, sl):
            return pltpu.make_async_copy(
                bufs.at[sl], out_hbm.at[:, pl.ds(wg * wc, wc)],
                out_sems.at[sl])

        if kfull > 0:
            in_copy(wid, 0).start()
        if kfull > 1:
            in_copy(wid + _NW, 1).start()

        vseg = b * d // _NS
        pltpu.sync_copy(vflat_hbm.at[pl.ds(sid * vseg, vseg)],
                        vals_sh.at[pl.ds(sid * vseg, vseg)])

        pltpu.sync_copy(idx_hbm, idx_v.at[pl.ds(0, b)])
        plsc.subcore_barrier()

        def scan_body(g, n):
            iv = idx_v[pl.ds(g * _L, _L)]
            q = lax.shift_right_logical(iv, shift)
            msk = (q & (_NW - 1)) == wid
            cnt = plsc.all_reduce_population_count(msk)[0]

            @pl.when((cnt > 0) & (n + cnt <= _CAP))
            def _():
                plsc.store_compressed(myj_v.at[pl.ds(n, _L)],
                                      lanes + g * _L, mask=msk)
                plsc.store_compressed(idx_v.at[pl.ds(n, _L)], iv, mask=msk)

            return n + cnt

        n = lax.fori_loop(0, b // _L, scan_body, jnp.int32(0), unroll=2)
        fb = n > _CAP

        @pl.when(~fb)
        def _():
            plsc.store_compressed(idx_v.at[pl.ds(n, _L)],
                                  jnp.full((_L,), _HUGE, dtype=jnp.int32),
                                  mask=full)

        @pl.when(fb)
        def _():
            pltpu.sync_copy(idx_hbm, idx_v.at[pl.ds(0, b)])

        nq = lax.div(n + _L - 1, _L)

        def apply_updates(pos_of, buf_store):
            def make_q_body(jv_of):
                def q_body(qi, _):
                    lv = idx_v[pl.ds(qi * _L, _L)]
                    pos, wm = pos_of(lv)
                    c = plsc.all_reduce_population_count(wm)[0]

                    @pl.when(c > 0)
                    def _():
                        plsc.store_compressed(tmpl_v.at[pl.ds(0, _L)], pos,
                                              mask=wm)
                        plsc.store_compressed(tmpj_v.at[pl.ds(0, _L)],
                                              jv_of(qi), mask=wm)

                        def fetch(t, carry):
                            jt = tmpj_v[pl.ds(t, _L)][0]
                            pltpu.make_async_copy(
                                vals_sh.at[pl.ds(jt * d, d)],
                                vslots.at[pl.ds(t * d, d)], gsem).start()
                            return carry

                        lax.fori_loop(0, c, fetch, jnp.int32(0))

                        def drain(t, carry):
                            pltpu.make_async_copy(
                                vals_sh.at[pl.ds(0, d)],
                                vslots.at[pl.ds(0, d)], gsem).wait()
                            return carry

                        lax.fori_loop(0, c, drain, jnp.int32(0))

                        def item(t, carry):
                            pt = tmpl_v[pl.ds(t, _L)][0]
                            buf_store(pt, t)
                            return carry

                        lax.fori_loop(0, c, item, jnp.int32(0))

                    return 0

                return q_body

            @pl.when(~fb)
            def _():
                lax.fori_loop(0, nq,
                              make_q_body(
                                  lambda qi: myj_v[pl.ds(qi * _L, _L)]),
                              0)

            @pl.when(fb)
            def _():
                lax.fori_loop(0, b // _L,
                              make_q_body(lambda qi: lanes + qi * _L),
                              0)

        def window_pos_of(wg):
            def pos_of(lv):
                wm = lax.shift_right_logical(lv, shift) == wg
                return lv & (wc - 1), wm
            return pos_of

        def col_store_in(sl):
            def col_store(pt, t):
                posv = jnp.full((_L,), pt, jnp.int32)
                slv = jnp.full((_L,), sl, jnp.int32)
                g0 = plsc.load_gather(bufs, [slv, lanes, posv])
                g1 = plsc.load_gather(bufs, [slv, lanes + _L, posv])
                plsc.store_scatter(bufs, [slv, lanes, posv],
                                   g0 + vslots[pl.ds(t * d, _L)])
                plsc.store_scatter(bufs, [slv, lanes + _L, posv],
                                   g1 + vslots[pl.ds(t * d + _L, _L)])
            return col_store

        def pipe_body(k, _):
            sl = k & 3
            wg = wid + _NW * k
            in_copy(wg, sl).wait()

            @pl.when(k + 2 < kfull)
            def _():
                sl2 = (k + 2) & 3

                @pl.when(k >= 2)
                def _():
                    out_copy(wg - 2 * _NW, sl2).wait()

                in_copy(wg + 2 * _NW, sl2).start()

            apply_updates(window_pos_of(wg), col_store_in(sl))
            out_copy(wg, sl).start()
            return 0

        lax.fori_loop(0, kfull, pipe_body, 0)
        for j in range(max(kfull - 4, 0), kfull):
            out_copy(wid + _NW * j, j & 3).wait()

        if nleft:
            @pl.when(wid < nleft)
            def _():
                wg = kfull * _NW + wid
                pltpu.sync_copy(xt_hbm.at[:, pl.ds(wg * wc, wc)], bufs.at[0])
                apply_updates(window_pos_of(wg), col_store_in(0))
                pltpu.sync_copy(bufs.at[0],
                                out_hbm.at[:, pl.ds(wg * wc, wc)])

        if rem_main:
            @pl.when(wid == tail_owner)
            def _():
                base = nfull * wc
                pltpu.sync_copy(xt_hbm.at[:, pl.ds(base, rem_main)],
                                bufs.at[0, :, pl.ds(0, rem_main)])

                def pos_of(lv):
                    wm = lax.shift_right_logical(lv, shift) == nfull
                    pos = lv & (wc - 1)
                    return pos, wm & (pos < rem_main)

                apply_updates(pos_of, col_store_in(0))
                pltpu.sync_copy(bufs.at[0, :, pl.ds(0, rem_main)],
                                out_hbm.at[:, pl.ds(base, rem_main)])

        if rem_tail:
            @pl.when(wid == tail_owner)
            def _():
                def row_store(pt, t):
                    for h in range(d // _L):
                        cur = btail[pt, pl.ds(h * _L, _L)]
                        btail[pt, pl.ds(h * _L, _L)] = (
                            cur + vslots[pl.ds(t * d + h * _L, _L)])

                pltpu.sync_copy(xtail_hbm, btail)

                def pos_of(lv):
                    wm = lax.shift_right_logical(lv, shift) == nfull
                    pos = (lv & (wc - 1)) - rem_main
                    return pos, wm & (pos >= 0)

                apply_updates(pos_of, row_store)
                pltpu.sync_copy(btail, tail_hbm)

    def run(x, indices, values):
        xt = jnp.swapaxes(x, 0, 1)
        vflat = values.reshape(-1)
        if rem_tail:
            xtail = lax.slice(x, (nfull * wc + rem_main, 0), (m, d))
            out_t, out_tail = scatter_kernel(xt, xtail, indices, vflat)
            out = jnp.swapaxes(out_t, 0, 1)
            return lax.dynamic_update_slice(out, out_tail,
                                            (nfull * wc + rem_main, 0))
        (out_t,) = scatter_kernel(xt, indices, vflat)
        return jnp.swapaxes(out_t, 0, 1)

    return run


def kernel(x, indices, values):
    m, d = x.shape
    b = indices.shape[0]
    fn = _make(m, d, b, wc=512, shift=9)
    return fn(x, indices, values)
